# Initial kernel scaffold; baseline (speedup 1.0000x reference)
import jax
import jax.numpy as jnp
from jax.experimental import pallas as pl


def kernel(T, input_scores, input_skills, mean, sigma, cognition_0, W_cog, W_pred, b_pred):
    raise NotImplementedError("write your pallas kernel here")


# speedup vs baseline: 135.3341x; 135.3341x over previous
import jax
import jax.numpy as jnp
from jax import lax
from jax.experimental import pallas as pl
from jax.experimental.pallas import tpu as pltpu
from jax.experimental.pallas import tpu_sc as plsc

_SC_CORES = 2
_SC_SUBCORES = 16
_SC_WORKERS = _SC_CORES * _SC_SUBCORES


_CH = 128


def _gather_body(table_hbm, idx_hbm, out_hbm, idx_v, rows_v, sem):
    b_per_w = idx_v.shape[0]
    n_chunks = b_per_w // _CH
    wid = lax.axis_index("s") * _SC_CORES + lax.axis_index("c")
    pltpu.sync_copy(idx_hbm.at[pl.ds(wid * b_per_w, b_per_w)], idx_v)
    for c in range(n_chunks):
        pltpu.async_copy(
            table_hbm.at[idx_v.at[pl.ds(c * _CH, _CH)]], rows_v, sem).wait()
        pltpu.sync_copy(
            rows_v, out_hbm.at[pl.ds(wid * b_per_w + c * _CH, _CH)])


def _sc_gather(table, idx):
    n, d = idx.shape[0], table.shape[1]
    b_per_w = n // _SC_WORKERS
    mesh = plsc.VectorSubcoreMesh(core_axis_name="c", subcore_axis_name="s")
    return pl.kernel(
        _gather_body,
        out_type=jax.ShapeDtypeStruct((n, d), table.dtype),
        mesh=mesh,
        scratch_types=[
            pltpu.VMEM((b_per_w,), jnp.int32),
            pltpu.VMEM((_CH, d), table.dtype),
            pltpu.SemaphoreType.DMA,
        ],
    )(table, idx)


def _fnn_body(scores_ref, skills_ref, a0_ref, wcT_ref, wp_ref, params_ref,
              cog_ref, pred_ref, aext_ref):
    t_steps = scores_ref.shape[0]
    cog_d = aext_ref.shape[1]
    term = wcT_ref.shape[1] // cog_d
    aext_ref[...] = a0_ref[...]
    wcT = wcT_ref[...]
    wp = wp_ref[...]
    b_pred = params_ref[2 * term]

    def step(t, _):
        prev = aext_ref[pl.ds(t, 1)][0]
        s_t = scores_ref[pl.ds(t, 1)][0]
        blocks = []
        for i in range(term):
            d = s_t - params_ref[i]
            sg = params_ref[term + i]
            fs = jnp.exp(-(d * d) / (sg * sg))
            blocks.append(fs * prev)
        u = jnp.concatenate(blocks, axis=0)
        cn = jnp.dot(wcT, u, preferred_element_type=jnp.float32,
                     precision=lax.Precision.HIGHEST)
        slot = cn / jnp.sum(cn, axis=0, keepdims=True)
        sk_t = skills_ref[pl.ds(t, 1)][0]
        eq = skills_ref[...] == sk_t[None]
        aext_ref[...] = jnp.where(eq, slot[None], aext_ref[...])
        cur = aext_ref[pl.ds(t + 1, 1)][0]
        cog_ref[pl.ds(t, 1)] = cur[None]
        p = jnp.dot(wp, cur, preferred_element_type=jnp.float32,
                    precision=lax.Precision.HIGHEST) + b_pred
        pred_ref[pl.ds(t, 1)] = jnp.clip(p, 0.0, 1.0)[None]
        return 0

    lax.fori_loop(0, t_steps, step, 0)


def kernel(T, input_scores, input_skills, mean, sigma, cognition_0, W_cog,
           W_pred, b_pred):
    b_sz, t_steps = input_scores.shape
    cog_d = cognition_0.shape[1]
    term = mean.shape[0]
    v = t_steps + 1

    skills = input_skills.astype(jnp.int32)
    idx_flat = skills.T.reshape(v * b_sz)
    n_pad = -(-(v * b_sz) // (_SC_WORKERS * _CH)) * (_SC_WORKERS * _CH)
    idx_pad = jnp.zeros((n_pad,), jnp.int32).at[: v * b_sz].set(idx_flat)
    table_pad = jnp.pad(cognition_0, ((0, 0), (0, 128 - cog_d)))
    g = _sc_gather(table_pad, idx_pad)
    a0 = (g[: v * b_sz, :cog_d]
          .reshape(v, b_sz, cog_d).transpose(0, 2, 1))

    scores3 = input_scores.T.reshape(t_steps, 1, b_sz)
    skills3 = skills.T.reshape(v, 1, b_sz)
    wcT = W_cog.T
    wp = W_pred.T
    params = (jnp.zeros((16,), jnp.float32)
              .at[0:term].set(mean)
              .at[term:2 * term].set(sigma)
              .at[2 * term].set(b_pred[0]))

    cog_t, pred_t = pl.pallas_call(
        _fnn_body,
        out_shape=[
            jax.ShapeDtypeStruct((t_steps, cog_d, b_sz), jnp.float32),
            jax.ShapeDtypeStruct((t_steps, 1, b_sz), jnp.float32),
        ],
        in_specs=[
            pl.BlockSpec(memory_space=pltpu.VMEM),
            pl.BlockSpec(memory_space=pltpu.VMEM),
            pl.BlockSpec(memory_space=pltpu.VMEM),
            pl.BlockSpec(memory_space=pltpu.VMEM),
            pl.BlockSpec(memory_space=pltpu.VMEM),
            pl.BlockSpec(memory_space=pltpu.SMEM),
        ],
        out_specs=[
            pl.BlockSpec(memory_space=pltpu.VMEM),
            pl.BlockSpec(memory_space=pltpu.VMEM),
        ],
        scratch_shapes=[pltpu.VMEM((v, cog_d, b_sz), jnp.float32)],
    )(scores3, skills3, a0, wcT, wp, params)

    pred = pred_t[:, 0, :].T
    cog_out = cog_t.transpose(2, 0, 1)
    valid = jnp.arange(t_steps) < T
    pred = jnp.where(valid[None, :], pred, 0.0)
    cog_out = jnp.where(valid[None, :, None], cog_out, 0.0)
    return (pred, cog_out)


# speedup vs baseline: 175.3714x; 1.2958x over previous
import jax
import jax.numpy as jnp
from jax import lax
from jax.experimental import pallas as pl
from jax.experimental.pallas import tpu as pltpu
from jax.experimental.pallas import tpu_sc as plsc

_SC_CORES = 2
_SC_SUBCORES = 16
_SC_WORKERS = _SC_CORES * _SC_SUBCORES


_CH = 128


_NBUF = 4


def _gather_body(table_hbm, idx_hbm, out_hbm, idx_v, rows, gsem, ssem):
    b_per_w = idx_v.shape[0]
    n_chunks = b_per_w // _CH
    wid = lax.axis_index("s") * _SC_CORES + lax.axis_index("c")
    base = wid * b_per_w
    pltpu.sync_copy(idx_hbm.at[pl.ds(base, b_per_w)], idx_v)
    gh = [None] * _NBUF
    sh = [None] * _NBUF

    def start_gather(c):
        b = c % _NBUF
        gh[b] = pltpu.async_copy(
            table_hbm.at[idx_v.at[pl.ds(c * _CH, _CH)]], rows.at[b],
            gsem.at[b])

    for c in range(min(_NBUF - 1, n_chunks)):
        start_gather(c)
    for c in range(n_chunks):
        b = c % _NBUF
        gh[b].wait()
        sh[b] = pltpu.async_copy(
            rows.at[b], out_hbm.at[pl.ds(base + c * _CH, _CH)], ssem.at[b])
        nc = c + _NBUF - 1
        if nc < n_chunks:
            nb = nc % _NBUF
            if sh[nb] is not None:
                sh[nb].wait()
                sh[nb] = None
            start_gather(nc)
    for b in range(_NBUF):
        if sh[b] is not None:
            sh[b].wait()


def _sc_gather(table, idx):
    n, d = idx.shape[0], table.shape[1]
    b_per_w = n // _SC_WORKERS
    mesh = plsc.VectorSubcoreMesh(core_axis_name="c", subcore_axis_name="s")
    return pl.kernel(
        _gather_body,
        out_type=jax.ShapeDtypeStruct((n, d), table.dtype),
        mesh=mesh,
        scratch_types=[
            pltpu.VMEM((b_per_w,), jnp.int32),
            pltpu.VMEM((_NBUF, _CH, d), table.dtype),
            pltpu.SemaphoreType.DMA((_NBUF,)),
            pltpu.SemaphoreType.DMA((_NBUF,)),
        ],
    )(table, idx)


def _fnn_body(scores_ref, skills_ref, a0_ref, wcT_ref, wp_ref, params_ref,
              cog_ref, pred_ref, aext_ref):
    t_steps = scores_ref.shape[0]
    cog_d = aext_ref.shape[1]
    term = wcT_ref.shape[1] // cog_d
    v_rows = aext_ref.shape[0]
    aext_ref[...] = a0_ref[...]
    wcT = wcT_ref[...]
    wp = wp_ref[...]
    b_pred = params_ref[2 * term]

    prev = aext_ref[0]
    for t in range(t_steps):
        s_t = scores_ref[t]
        blocks = []
        for i in range(term):
            d = s_t - params_ref[i]
            sg = params_ref[term + i]
            fs = jnp.exp(-(d * d) / (sg * sg))
            blocks.append(fs * prev)
        u = jnp.concatenate(blocks, axis=0)
        cn = jnp.dot(wcT, u, preferred_element_type=jnp.float32,
                     precision=lax.Precision.HIGHEST)
        slot = cn / jnp.sum(cn, axis=0, keepdims=True)
        tail = v_rows - (t + 1)
        sk_t = skills_ref[t]
        eq = skills_ref[pl.ds(t + 1, tail)] == sk_t[None]
        aext_ref[pl.ds(t + 1, tail)] = jnp.where(
            eq, slot[None], aext_ref[pl.ds(t + 1, tail)])
        cur = aext_ref[t + 1]
        cog_ref[t] = cur
        p = jnp.dot(wp, cur, preferred_element_type=jnp.float32,
                    precision=lax.Precision.HIGHEST) + b_pred
        pred_ref[t] = jnp.clip(p, 0.0, 1.0)
        prev = cur


def kernel(T, input_scores, input_skills, mean, sigma, cognition_0, W_cog,
           W_pred, b_pred):
    b_sz, t_steps = input_scores.shape
    cog_d = cognition_0.shape[1]
    term = mean.shape[0]
    v = t_steps + 1

    skills = input_skills.astype(jnp.int32)
    idx_flat = skills.T.reshape(v * b_sz)
    n_pad = -(-(v * b_sz) // (_SC_WORKERS * _CH)) * (_SC_WORKERS * _CH)
    idx_pad = jnp.zeros((n_pad,), jnp.int32).at[: v * b_sz].set(idx_flat)
    table_pad = jnp.pad(cognition_0, ((0, 0), (0, 128 - cog_d)))
    g = _sc_gather(table_pad, idx_pad)
    a0 = (g[: v * b_sz, :cog_d]
          .reshape(v, b_sz, cog_d).transpose(0, 2, 1))

    scores3 = input_scores.T.reshape(t_steps, 1, b_sz)
    skills3 = skills.T.reshape(v, 1, b_sz)
    wcT = W_cog.T
    wp = W_pred.T
    params = (jnp.zeros((16,), jnp.float32)
              .at[0:term].set(mean)
              .at[term:2 * term].set(sigma)
              .at[2 * term].set(b_pred[0]))

    cog_t, pred_t = pl.pallas_call(
        _fnn_body,
        out_shape=[
            jax.ShapeDtypeStruct((t_steps, cog_d, b_sz), jnp.float32),
            jax.ShapeDtypeStruct((t_steps, 1, b_sz), jnp.float32),
        ],
        in_specs=[
            pl.BlockSpec(memory_space=pltpu.VMEM),
            pl.BlockSpec(memory_space=pltpu.VMEM),
            pl.BlockSpec(memory_space=pltpu.VMEM),
            pl.BlockSpec(memory_space=pltpu.VMEM),
            pl.BlockSpec(memory_space=pltpu.VMEM),
            pl.BlockSpec(memory_space=pltpu.SMEM),
        ],
        out_specs=[
            pl.BlockSpec(memory_space=pltpu.VMEM),
            pl.BlockSpec(memory_space=pltpu.VMEM),
        ],
        scratch_shapes=[pltpu.VMEM((v, cog_d, b_sz), jnp.float32)],
    )(scores3, skills3, a0, wcT, wp, params)

    pred = pred_t[:, 0, :].T
    cog_out = cog_t.transpose(2, 0, 1)
    valid = jnp.arange(t_steps) < T
    pred = jnp.where(valid[None, :], pred, 0.0)
    cog_out = jnp.where(valid[None, :, None], cog_out, 0.0)
    return (pred, cog_out)


# speedup vs baseline: 223.0288x; 1.2718x over previous
import jax
import jax.numpy as jnp
from jax import lax
from jax.experimental import pallas as pl
from jax.experimental.pallas import tpu as pltpu
from jax.experimental.pallas import tpu_sc as plsc

_SC_CORES = 2
_SC_SUBCORES = 16
_SC_WORKERS = _SC_CORES * _SC_SUBCORES


_CH = 128


_NBUF = 4


def _gather_body(table_hbm, idx_hbm, out_hbm, idx_v, rows, gsem, ssem):
    b_per_w = idx_v.shape[0]
    n_chunks = b_per_w // _CH
    wid = lax.axis_index("s") * _SC_CORES + lax.axis_index("c")
    base = wid * b_per_w
    pltpu.sync_copy(idx_hbm.at[pl.ds(base, b_per_w)], idx_v)
    gh = [None] * _NBUF
    sh = [None] * _NBUF

    def start_gather(c):
        b = c % _NBUF
        gh[b] = pltpu.async_copy(
            table_hbm.at[idx_v.at[pl.ds(c * _CH, _CH)]], rows.at[b],
            gsem.at[b])

    for c in range(min(_NBUF - 1, n_chunks)):
        start_gather(c)
    for c in range(n_chunks):
        b = c % _NBUF
        gh[b].wait()
        sh[b] = pltpu.async_copy(
            rows.at[b], out_hbm.at[pl.ds(base + c * _CH, _CH)], ssem.at[b])
        nc = c + _NBUF - 1
        if nc < n_chunks:
            nb = nc % _NBUF
            if sh[nb] is not None:
                sh[nb].wait()
                sh[nb] = None
            start_gather(nc)
    for b in range(_NBUF):
        if sh[b] is not None:
            sh[b].wait()


def _sc_gather(table, idx):
    n, d = idx.shape[0], table.shape[1]
    b_per_w = n // _SC_WORKERS
    mesh = plsc.VectorSubcoreMesh(core_axis_name="c", subcore_axis_name="s")
    return pl.kernel(
        _gather_body,
        out_type=jax.ShapeDtypeStruct((n, d), table.dtype),
        mesh=mesh,
        scratch_types=[
            pltpu.VMEM((b_per_w,), jnp.int32),
            pltpu.VMEM((_NBUF, _CH, d), table.dtype),
            pltpu.SemaphoreType.DMA((_NBUF,)),
            pltpu.SemaphoreType.DMA((_NBUF,)),
        ],
        compiler_params=pltpu.CompilerParams(use_tc_tiling_on_sc=False),
    )(table, idx)


def _fnn_body(scores_ref, skills_ref, a0_ref, wcT_ref, wp_ref, params_ref,
              cog_ref, pred_ref, aext_ref):
    t_steps = scores_ref.shape[0]
    cog_d = aext_ref.shape[1]
    term = wcT_ref.shape[1] // cog_d
    v_rows = aext_ref.shape[0]
    aext_ref[...] = a0_ref[...]
    wcT = wcT_ref[...]
    wp = wp_ref[...]
    b_pred = params_ref[2 * term]

    prev = aext_ref[0]
    for t in range(t_steps):
        s_t = scores_ref[t]
        blocks = []
        for i in range(term):
            d = s_t - params_ref[i]
            sg = params_ref[term + i]
            fs = jnp.exp(-(d * d) / (sg * sg))
            blocks.append(fs * prev)
        u = jnp.concatenate(blocks, axis=0)
        cn = jnp.dot(wcT, u, preferred_element_type=jnp.float32,
                     precision=lax.Precision.HIGHEST)
        slot = cn / jnp.sum(cn, axis=0, keepdims=True)
        tail = v_rows - (t + 1)
        sk_t = skills_ref[t]
        eq = skills_ref[pl.ds(t + 1, tail)] == sk_t[None]
        aext_ref[pl.ds(t + 1, tail)] = jnp.where(
            eq, slot[None], aext_ref[pl.ds(t + 1, tail)])
        cur = aext_ref[t + 1]
        cog_ref[t] = cur
        p = jnp.dot(wp, cur, preferred_element_type=jnp.float32,
                    precision=lax.Precision.HIGHEST) + b_pred
        pred_ref[t] = jnp.clip(p, 0.0, 1.0)
        prev = cur


def kernel(T, input_scores, input_skills, mean, sigma, cognition_0, W_cog,
           W_pred, b_pred):
    b_sz, t_steps = input_scores.shape
    cog_d = cognition_0.shape[1]
    term = mean.shape[0]
    v = t_steps + 1

    skills = input_skills.astype(jnp.int32)
    idx_flat = skills.T.reshape(v * b_sz)
    n_pad = -(-(v * b_sz) // (_SC_WORKERS * _CH)) * (_SC_WORKERS * _CH)
    idx_pad = jnp.zeros((n_pad,), jnp.int32).at[: v * b_sz].set(idx_flat)
    g = _sc_gather(cognition_0, idx_pad)
    a0 = (g[: v * b_sz]
          .reshape(v, b_sz, cog_d).transpose(0, 2, 1))

    scores3 = input_scores.T.reshape(t_steps, 1, b_sz)
    skills3 = skills.T.reshape(v, 1, b_sz)
    wcT = W_cog.T
    wp = W_pred.T
    params = (jnp.zeros((16,), jnp.float32)
              .at[0:term].set(mean)
              .at[term:2 * term].set(sigma)
              .at[2 * term].set(b_pred[0]))

    cog_t, pred_t = pl.pallas_call(
        _fnn_body,
        out_shape=[
            jax.ShapeDtypeStruct((t_steps, cog_d, b_sz), jnp.float32),
            jax.ShapeDtypeStruct((t_steps, 1, b_sz), jnp.float32),
        ],
        in_specs=[
            pl.BlockSpec(memory_space=pltpu.VMEM),
            pl.BlockSpec(memory_space=pltpu.VMEM),
            pl.BlockSpec(memory_space=pltpu.VMEM),
            pl.BlockSpec(memory_space=pltpu.VMEM),
            pl.BlockSpec(memory_space=pltpu.VMEM),
            pl.BlockSpec(memory_space=pltpu.SMEM),
        ],
        out_specs=[
            pl.BlockSpec(memory_space=pltpu.VMEM),
            pl.BlockSpec(memory_space=pltpu.VMEM),
        ],
        scratch_shapes=[pltpu.VMEM((v, cog_d, b_sz), jnp.float32)],
    )(scores3, skills3, a0, wcT, wp, params)

    pred = pred_t[:, 0, :].T
    cog_out = cog_t.transpose(2, 0, 1)
    valid = jnp.arange(t_steps) < T
    pred = jnp.where(valid[None, :], pred, 0.0)
    cog_out = jnp.where(valid[None, :, None], cog_out, 0.0)
    return (pred, cog_out)


# speedup vs baseline: 232.9728x; 1.0446x over previous
import jax
import jax.numpy as jnp
from jax import lax
from jax.experimental import pallas as pl
from jax.experimental.pallas import tpu as pltpu
from jax.experimental.pallas import tpu_sc as plsc

_SC_CORES = 2
_SC_SUBCORES = 16
_SC_WORKERS = _SC_CORES * _SC_SUBCORES


_CH = 128


_NBUF = 8


def _gather_body(table_hbm, idx_hbm, out_hbm, idx_v, rows, gsem, ssem):
    b_per_w = idx_v.shape[0]
    n_chunks = b_per_w // _CH
    wid = lax.axis_index("s") * _SC_CORES + lax.axis_index("c")
    base = wid * b_per_w
    pltpu.sync_copy(idx_hbm.at[pl.ds(base, b_per_w)], idx_v)
    gh = [None] * _NBUF
    sh = [None] * _NBUF

    def start_gather(c):
        b = c % _NBUF
        gh[b] = pltpu.async_copy(
            table_hbm.at[idx_v.at[pl.ds(c * _CH, _CH)]], rows.at[b],
            gsem.at[b])

    for c in range(min(_NBUF - 1, n_chunks)):
        start_gather(c)
    for c in range(n_chunks):
        b = c % _NBUF
        gh[b].wait()
        sh[b] = pltpu.async_copy(
            rows.at[b], out_hbm.at[pl.ds(base + c * _CH, _CH)], ssem.at[b])
        nc = c + _NBUF - 1
        if nc < n_chunks:
            nb = nc % _NBUF
            if sh[nb] is not None:
                sh[nb].wait()
                sh[nb] = None
            start_gather(nc)
    for b in range(_NBUF):
        if sh[b] is not None:
            sh[b].wait()


def _sc_gather(table, idx):
    n, d = idx.shape[0], table.shape[1]
    b_per_w = n // _SC_WORKERS
    mesh = plsc.VectorSubcoreMesh(core_axis_name="c", subcore_axis_name="s")
    return pl.kernel(
        _gather_body,
        out_type=jax.ShapeDtypeStruct((n, d), table.dtype),
        mesh=mesh,
        scratch_types=[
            pltpu.VMEM((b_per_w,), jnp.int32),
            pltpu.VMEM((_NBUF, _CH, d), table.dtype),
            pltpu.SemaphoreType.DMA((_NBUF,)),
            pltpu.SemaphoreType.DMA((_NBUF,)),
        ],
        compiler_params=pltpu.CompilerParams(use_tc_tiling_on_sc=False),
    )(table, idx)


def _fnn_body(scores_ref, skills_ref, a0_ref, wcT_ref, wp_ref, params_ref,
              tlim_ref, cog_ref, pred_ref, aext_ref):
    t_steps = scores_ref.shape[0]
    cog_d = aext_ref.shape[1]
    term = wcT_ref.shape[1] // cog_d
    v_rows = aext_ref.shape[0]
    aext_ref[...] = a0_ref[...]
    wcT = wcT_ref[...]
    wp = wp_ref[...]
    b_pred = params_ref[2 * term]
    tlim = tlim_ref[0]

    prev = aext_ref[0]
    for t in range(t_steps):
        s_t = scores_ref[t]
        blocks = []
        for i in range(term):
            d = s_t - params_ref[i]
            sg = params_ref[term + i]
            fs = jnp.exp(-(d * d) / (sg * sg))
            blocks.append(fs * prev)
        u = jnp.concatenate(blocks, axis=0)
        cn = jnp.dot(wcT, u, preferred_element_type=jnp.float32,
                     precision=lax.Precision.HIGHEST)
        slot = cn / jnp.sum(cn, axis=0, keepdims=True)
        tail = v_rows - (t + 1)
        sk_t = skills_ref[t]
        eq = skills_ref[pl.ds(t + 1, tail)] == sk_t[None]
        aext_ref[pl.ds(t + 1, tail)] = jnp.where(
            eq, slot[None], aext_ref[pl.ds(t + 1, tail)])
        cur = aext_ref[t + 1]
        live = t < tlim
        cog_ref[t] = jnp.where(live, cur, 0.0)
        p = jnp.dot(wp, cur, preferred_element_type=jnp.float32,
                    precision=lax.Precision.HIGHEST) + b_pred
        pred_ref[t] = jnp.where(live, jnp.clip(p, 0.0, 1.0), 0.0)
        prev = cur


def kernel(T, input_scores, input_skills, mean, sigma, cognition_0, W_cog,
           W_pred, b_pred):
    b_sz, t_steps = input_scores.shape
    cog_d = cognition_0.shape[1]
    term = mean.shape[0]
    v = t_steps + 1

    skills = input_skills.astype(jnp.int32)
    idx_flat = skills.T.reshape(v * b_sz)
    n_pad = -(-(v * b_sz) // (_SC_WORKERS * _CH)) * (_SC_WORKERS * _CH)
    idx_pad = jnp.zeros((n_pad,), jnp.int32).at[: v * b_sz].set(idx_flat)
    g = _sc_gather(cognition_0, idx_pad)
    a0 = (g[: v * b_sz]
          .reshape(v, b_sz, cog_d).transpose(0, 2, 1))

    scores3 = input_scores.T.reshape(t_steps, 1, b_sz)
    skills3 = skills.T.reshape(v, 1, b_sz)
    wcT = W_cog.T
    wp = W_pred.T
    params = (jnp.zeros((16,), jnp.float32)
              .at[0:term].set(mean)
              .at[term:2 * term].set(sigma)
              .at[2 * term].set(b_pred[0]))
    tlim = jnp.asarray(T, jnp.int32).reshape((1,))

    cog_t, pred_t = pl.pallas_call(
        _fnn_body,
        out_shape=[
            jax.ShapeDtypeStruct((t_steps, cog_d, b_sz), jnp.float32),
            jax.ShapeDtypeStruct((t_steps, 1, b_sz), jnp.float32),
        ],
        in_specs=[
            pl.BlockSpec(memory_space=pltpu.VMEM),
            pl.BlockSpec(memory_space=pltpu.VMEM),
            pl.BlockSpec(memory_space=pltpu.VMEM),
            pl.BlockSpec(memory_space=pltpu.VMEM),
            pl.BlockSpec(memory_space=pltpu.VMEM),
            pl.BlockSpec(memory_space=pltpu.SMEM),
            pl.BlockSpec(memory_space=pltpu.SMEM),
        ],
        out_specs=[
            pl.BlockSpec(memory_space=pltpu.VMEM),
            pl.BlockSpec(memory_space=pltpu.VMEM),
        ],
        scratch_shapes=[pltpu.VMEM((v, cog_d, b_sz), jnp.float32)],
    )(scores3, skills3, a0, wcT, wp, params, tlim)

    pred = pred_t[:, 0, :].T
    cog_out = cog_t.transpose(2, 0, 1)
    return (pred, cog_out)
